# baseline (speedup 1.0000x reference)
import jax
import jax.numpy as jnp
from jax import lax
from jax.experimental import pallas as pl
from jax.experimental.pallas import tpu as pltpu, tpu_sc as plsc

N = 10000
D = 128
H = D // 2
E = 320000

NC = 2
NS = 16
NW = NC * NS

C = 128
NCH = 160
EPE = E // NS
EPT = NCH * C
NBUF = 4
NIB = 8
NGRP = NCH // NIB
NPAD = 10240
RPT = NPAD // NS
CO = 128
NCO = RPT // CO

_mesh = plsc.VectorSubcoreMesh(
    core_axis_name="c", subcore_axis_name="s", num_cores=NC, num_subcores=NS
)


def _sc_agg_body(with_deg, *refs):
    if with_deg:
        (zs, srcr, dstr, zeros_a, zeros_d, ones_h, outp, outd,
         cbuf, acc, ones_v, dbuf, dacc) = refs[:13]
        rest = refs[13:]
    else:
        (zs, srcr, dstr, zeros_a, outp, cbuf, acc) = refs[:7]
        rest = refs[7:]
    sib = rest[:NIB]
    dib = rest[NIB:2 * NIB]
    rows = rest[2 * NIB:2 * NIB + NBUF]
    ssem = rest[2 * NIB + NBUF:3 * NIB + NBUF]
    dsem = rest[3 * NIB + NBUF:4 * NIB + NBUF]
    gsem = rest[4 * NIB + NBUF:]

    c = lax.axis_index("c")
    s = lax.axis_index("s")

    tbl = zs.at[pl.ds(c, 2 * N - 1)]
    srcs = srcr.at[s]
    dsts = dstr.at[s]
    for ib in range(NIB):
        pltpu.make_async_copy(srcs.at[ib], sib[ib], ssem[ib]).start()
        pltpu.make_async_copy(dsts.at[ib], dib[ib], dsem[ib]).start()
    for b in range(NBUF):
        pltpu.make_async_copy(srcs.at[b], sib[b], ssem[b]).wait()
        pltpu.make_async_copy(tbl.at[sib[b]], rows[b], gsem[b]).start()

    pltpu.sync_copy(zeros_a, cbuf)
    for t in range(NCO):
        pltpu.sync_copy(cbuf, acc.at[pl.ds(s * RPT + t * CO, CO)])
    if with_deg:
        pltpu.sync_copy(ones_h, ones_v)
        pltpu.sync_copy(zeros_d, dbuf)
        pltpu.sync_copy(dbuf, dacc.at[pl.ds(s * RPT, RPT)])
    plsc.subcore_barrier()

    def group(g, carry):
        for k in range(NIB):
            j = g * NIB + k
            b = k % NBUF
            pltpu.make_async_copy(
                tbl.at[sib[k]], rows[b], gsem[b]).wait()
            pltpu.make_async_copy(dsts.at[j], dib[k], dsem[k]).wait()
            pltpu.sync_copy(rows[b], acc.at[dib[k]], add=True)
            if with_deg:
                @pl.when(c == (k % 2))
                def _():
                    pltpu.sync_copy(ones_v, dacc.at[dib[k]], add=True)

            @pl.when(g < NGRP - 1)
            def _():
                pltpu.make_async_copy(
                    srcs.at[j + NIB], sib[k], ssem[k]).start()
                pltpu.make_async_copy(
                    dsts.at[j + NIB], dib[k], dsem[k]).start()

            kn = (k + NBUF) % NIB

            @pl.when(j + NBUF < NCH)
            def _():
                pltpu.make_async_copy(
                    srcs.at[j + NBUF], sib[kn], ssem[kn]).wait()
                pltpu.make_async_copy(
                    tbl.at[sib[kn]], rows[b], gsem[b]).start()
        return carry

    lax.fori_loop(0, NGRP, group, 0)
    plsc.subcore_barrier()

    for t in range(NCO):
        row0 = s * RPT + t * CO
        pltpu.sync_copy(acc.at[pl.ds(row0, CO)], cbuf)
        pltpu.sync_copy(cbuf, outp.at[pl.ds(row0, CO), pl.ds(c * H, H)])
    if with_deg:
        pltpu.sync_copy(dacc.at[pl.ds(s * RPT, RPT)], dbuf)
        pltpu.sync_copy(dbuf, outd.at[c, pl.ds(s * RPT, RPT)])


_ring_scratch = (
    [pltpu.VMEM((C,), jnp.int32)] * (2 * NIB)
    + [pltpu.VMEM((C, H), jnp.float32)] * NBUF
    + [pltpu.SemaphoreType.DMA] * (2 * NIB + NBUF)
)

_common_scratch = [
    pltpu.VMEM((CO, H), jnp.float32),
    pltpu.VMEM_SHARED((NPAD, H), jnp.float32),
]


def _agg_deg_body(*refs):
    _sc_agg_body(True, *refs)


def _agg_body(*refs):
    _sc_agg_body(False, *refs)


_sc_params = pltpu.CompilerParams(use_tc_tiling_on_sc=False)


def _prep_body(ei_ref, s_ref, d_ref):
    pad_dst = (lax.broadcasted_iota(jnp.int32, (1, EPT - EPE), 1) % 128
               + (NPAD - 128))
    zpad = jnp.zeros((1, EPT - EPE), jnp.int32)
    for s in range(NS):
        sl = slice(s * EPE, (s + 1) * EPE)
        s_ref[s:s + 1, :EPE] = ei_ref[0:1, sl] * 2
        d_ref[s:s + 1, :EPE] = ei_ref[1:2, sl]
        s_ref[s:s + 1, EPE:] = zpad
        d_ref[s:s + 1, EPE:] = pad_dst


def _prep(edge_index):
    out = pl.pallas_call(
        _prep_body,
        in_specs=[pl.BlockSpec((2, E), lambda: (0, 0))],
        out_specs=[pl.BlockSpec((NS, EPT), lambda: (0, 0)),
                   pl.BlockSpec((NS, EPT), lambda: (0, 0))],
        grid=(),
        out_shape=[jax.ShapeDtypeStruct((NS, EPT), jnp.int32),
                   jax.ShapeDtypeStruct((NS, EPT), jnp.int32)],
    )(edge_index)
    return out[0].reshape(NS, NCH, C), out[1].reshape(NS, NCH, C)

_agg_deg = pl.kernel(
    _agg_deg_body,
    compiler_params=_sc_params,
    out_type=[
        jax.ShapeDtypeStruct((NPAD, D), jnp.float32),
        jax.ShapeDtypeStruct((NC, NPAD, 16), jnp.float32),
    ],
    mesh=_mesh,
    scratch_types=_common_scratch + [
        pltpu.VMEM((C, 16), jnp.float32),
        pltpu.VMEM((RPT, 16), jnp.float32),
        pltpu.VMEM_SHARED((NPAD, 16), jnp.float32),
    ] + _ring_scratch,
)

_agg = pl.kernel(
    _agg_body,
    compiler_params=_sc_params,
    out_type=jax.ShapeDtypeStruct((NPAD, D), jnp.float32),
    mesh=_mesh,
    scratch_types=_common_scratch + _ring_scratch,
)


BR = 2000


def _dense_body(z_ref, p_ref, d0_ref, d1_ref, w_ref, o_ref):
    deg = jnp.maximum((d0_ref[...] + d1_ref[...])[:, 0:1], 1.0)
    zn = p_ref[...] / deg
    zl = z_ref[...]
    wl = w_ref[:, :D]
    wr = w_ref[:, D:]
    dn = (((1,), (1,)), ((), ()))
    acc = lax.dot_general(zl, wl, dn, preferred_element_type=jnp.float32)
    acc = acc + lax.dot_general(zn, wr, dn, preferred_element_type=jnp.float32)
    h = jax.nn.sigmoid(acc)
    norm = jnp.sqrt(jnp.sum(h * h, axis=1, keepdims=True)) + 1e-12
    o_ref[...] = h / norm


def _dense(z, p, d, W):
    def body(z_ref, p_ref, d0_ref, d1_ref, w_ref, o_ref):
        _dense_body(z_ref, p_ref, d0_ref.at[0], d1_ref.at[0], w_ref, o_ref)

    return pl.pallas_call(
        body,
        grid=(N // BR,),
        in_specs=[
            pl.BlockSpec((BR, D), lambda i: (i, 0)),
            pl.BlockSpec((BR, D), lambda i: (i, 0)),
            pl.BlockSpec((1, BR, 16), lambda i: (0, i, 0)),
            pl.BlockSpec((1, BR, 16), lambda i: (1, i, 0)),
            pl.BlockSpec((D, 2 * D), lambda i: (0, 0)),
        ],
        out_specs=pl.BlockSpec((BR, D), lambda i: (i, 0)),
        out_shape=jax.ShapeDtypeStruct((N, D), jnp.float32),
        compiler_params=pltpu.CompilerParams(
            dimension_semantics=("parallel",)),
    )(z, p, d, d, W)


@jax.jit
def kernel(x, edge_index, W1, W2):
    srcr, dstr = _prep(edge_index.astype(jnp.int32))
    zeros_a = jnp.zeros((CO, H), jnp.float32)
    zeros_d = jnp.zeros((RPT, 16), jnp.float32)
    ones_h = jnp.ones((C, 16), jnp.float32)

    xflat = x.reshape(2 * N, H)
    p, dp = _agg_deg(xflat, srcr, dstr, zeros_a, zeros_d, ones_h)
    z1 = _dense(x, p, dp, W1)
    p2 = _agg(z1.reshape(2 * N, H), srcr, dstr, zeros_a)
    z2 = _dense(z1, p2, dp, W2)
    return z2


# speedup vs baseline: 3.1800x; 3.1800x over previous
import jax
import jax.numpy as jnp
from jax import lax
from jax.experimental import pallas as pl
from jax.experimental.pallas import tpu as pltpu, tpu_sc as plsc

N = 10000
D = 128
H = D // 2
E = 320000

NC = 2
NS = 16
NW = NC * NS

C = 125
NCH = E // NS // C
NBUF = 4
NIB = 8
NGRP = NCH // NIB
NPAD = 10240
RPT = NPAD // NS
CO = 128
NCO = RPT // CO

_mesh = plsc.VectorSubcoreMesh(
    core_axis_name="c", subcore_axis_name="s", num_cores=NC, num_subcores=NS
)


def _sc_agg_body(with_deg, *refs):
    if with_deg:
        (zs, srcr, dstr, zeros_a, zeros_d, ones_h, outp, outd,
         cbuf, acc, ones_v, dbuf, dacc) = refs[:13]
        rest = refs[13:]
    else:
        (zs, srcr, dstr, zeros_a, outp, cbuf, acc) = refs[:7]
        rest = refs[7:]
    sib = rest[:NIB]
    dib = rest[NIB:2 * NIB]
    rows = rest[2 * NIB:2 * NIB + NBUF]
    ssem = rest[2 * NIB + NBUF:3 * NIB + NBUF]
    dsem = rest[3 * NIB + NBUF:4 * NIB + NBUF]
    gsem = rest[4 * NIB + NBUF:]

    c = lax.axis_index("c")
    s = lax.axis_index("s")

    tbl = zs.at[pl.ds(c, 2 * N - 1)]
    srcs = srcr.at[s]
    dsts = dstr.at[s]
    for ib in range(NIB):
        pltpu.make_async_copy(srcs.at[ib], sib[ib], ssem[ib]).start()
        pltpu.make_async_copy(dsts.at[ib], dib[ib], dsem[ib]).start()
    for b in range(NBUF):
        pltpu.make_async_copy(srcs.at[b], sib[b], ssem[b]).wait()
        pltpu.make_async_copy(tbl.at[sib[b]], rows[b], gsem[b]).start()

    pltpu.sync_copy(zeros_a, cbuf)
    for t in range(NCO):
        pltpu.sync_copy(cbuf, acc.at[pl.ds(s * RPT + t * CO, CO)])
    if with_deg:
        pltpu.sync_copy(ones_h, ones_v)
        pltpu.sync_copy(zeros_d, dbuf)
        pltpu.sync_copy(dbuf, dacc.at[pl.ds(s * RPT, RPT)])
    plsc.subcore_barrier()

    def group(g, carry):
        for k in range(NIB):
            j = g * NIB + k
            b = k % NBUF
            pltpu.make_async_copy(
                tbl.at[sib[k]], rows[b], gsem[b]).wait()
            pltpu.make_async_copy(dsts.at[j], dib[k], dsem[k]).wait()
            pltpu.sync_copy(rows[b], acc.at[dib[k]], add=True)
            if with_deg:
                @pl.when(c == (k % 2))
                def _():
                    pltpu.sync_copy(ones_v, dacc.at[dib[k]], add=True)

            @pl.when(g < NGRP - 1)
            def _():
                pltpu.make_async_copy(
                    srcs.at[j + NIB], sib[k], ssem[k]).start()
                pltpu.make_async_copy(
                    dsts.at[j + NIB], dib[k], dsem[k]).start()

            kn = (k + NBUF) % NIB

            @pl.when(j + NBUF < NCH)
            def _():
                pltpu.make_async_copy(
                    srcs.at[j + NBUF], sib[kn], ssem[kn]).wait()
                pltpu.make_async_copy(
                    tbl.at[sib[kn]], rows[b], gsem[b]).start()
        return carry

    lax.fori_loop(0, NGRP, group, 0)
    plsc.subcore_barrier()

    for t in range(NCO):
        row0 = s * RPT + t * CO
        pltpu.sync_copy(acc.at[pl.ds(row0, CO)], cbuf)
        pltpu.sync_copy(cbuf, outp.at[pl.ds(row0, CO), pl.ds(c * H, H)])
    if with_deg:
        pltpu.sync_copy(dacc.at[pl.ds(s * RPT, RPT)], dbuf)
        pltpu.sync_copy(dbuf, outd.at[c, pl.ds(s * RPT, RPT)])


_ring_scratch = (
    [pltpu.VMEM((C,), jnp.int32)] * (2 * NIB)
    + [pltpu.VMEM((C, H), jnp.float32)] * NBUF
    + [pltpu.SemaphoreType.DMA] * (2 * NIB + NBUF)
)

_common_scratch = [
    pltpu.VMEM((CO, H), jnp.float32),
    pltpu.VMEM_SHARED((NPAD, H), jnp.float32),
]


def _agg_deg_body(*refs):
    _sc_agg_body(True, *refs)


def _agg_body(*refs):
    _sc_agg_body(False, *refs)


_sc_params = pltpu.CompilerParams(use_tc_tiling_on_sc=False)


_agg_deg = pl.kernel(
    _agg_deg_body,
    compiler_params=_sc_params,
    out_type=[
        jax.ShapeDtypeStruct((NPAD, D), jnp.float32),
        jax.ShapeDtypeStruct((NC, NPAD, 16), jnp.float32),
    ],
    mesh=_mesh,
    scratch_types=_common_scratch + [
        pltpu.VMEM((C, 16), jnp.float32),
        pltpu.VMEM((RPT, 16), jnp.float32),
        pltpu.VMEM_SHARED((NPAD, 16), jnp.float32),
    ] + _ring_scratch,
)

_agg = pl.kernel(
    _agg_body,
    compiler_params=_sc_params,
    out_type=jax.ShapeDtypeStruct((NPAD, D), jnp.float32),
    mesh=_mesh,
    scratch_types=_common_scratch + _ring_scratch,
)


BR = 2000


def _dense_body(z_ref, p_ref, d0_ref, d1_ref, w_ref, o_ref):
    deg = jnp.maximum((d0_ref[...] + d1_ref[...])[:, 0:1], 1.0)
    zn = p_ref[...] / deg
    zl = z_ref[...]
    wl = w_ref[:, :D]
    wr = w_ref[:, D:]
    dn = (((1,), (1,)), ((), ()))
    acc = lax.dot_general(zl, wl, dn, preferred_element_type=jnp.float32)
    acc = acc + lax.dot_general(zn, wr, dn, preferred_element_type=jnp.float32)
    h = jax.nn.sigmoid(acc)
    norm = jnp.sqrt(jnp.sum(h * h, axis=1, keepdims=True)) + 1e-12
    o_ref[...] = h / norm


def _dense(z, p, d, W):
    def body(z_ref, p_ref, d0_ref, d1_ref, w_ref, o_ref):
        _dense_body(z_ref, p_ref, d0_ref.at[0], d1_ref.at[0], w_ref, o_ref)

    return pl.pallas_call(
        body,
        grid=(N // BR,),
        in_specs=[
            pl.BlockSpec((BR, D), lambda i: (i, 0)),
            pl.BlockSpec((BR, D), lambda i: (i, 0)),
            pl.BlockSpec((1, BR, 16), lambda i: (0, i, 0)),
            pl.BlockSpec((1, BR, 16), lambda i: (1, i, 0)),
            pl.BlockSpec((D, 2 * D), lambda i: (0, 0)),
        ],
        out_specs=pl.BlockSpec((BR, D), lambda i: (i, 0)),
        out_shape=jax.ShapeDtypeStruct((N, D), jnp.float32),
        compiler_params=pltpu.CompilerParams(
            dimension_semantics=("parallel",)),
    )(z, p, d, d, W)


@jax.jit
def kernel(x, edge_index, W1, W2):
    srcr = (edge_index[0].astype(jnp.int32) * 2).reshape(NS, NCH, C)
    dstr = edge_index[1].astype(jnp.int32).reshape(NS, NCH, C)
    zeros_a = jnp.zeros((CO, H), jnp.float32)
    zeros_d = jnp.zeros((RPT, 16), jnp.float32)
    ones_h = jnp.ones((C, 16), jnp.float32)

    xflat = x.reshape(2 * N, H)
    p, dp = _agg_deg(xflat, srcr, dstr, zeros_a, zeros_d, ones_h)
    z1 = _dense(x, p, dp, W1)
    p2 = _agg(z1.reshape(2 * N, H), srcr, dstr, zeros_a)
    z2 = _dense(z1, p2, dp, W2)
    return z2


# speedup vs baseline: 3.6531x; 1.1488x over previous
import jax
import jax.numpy as jnp
from jax import lax
from jax.experimental import pallas as pl
from jax.experimental.pallas import tpu as pltpu, tpu_sc as plsc

N = 10000
D = 128
H = D // 2
E = 320000

NC = 2
NS = 16
NW = NC * NS

C = 125
NCH = E // NS // C
NBUF = 4
NIB = 8
NGRP = NCH // NIB
NPAD = 10240
RPT = NPAD // NS
CO = 128
NCO = RPT // CO

_mesh = plsc.VectorSubcoreMesh(
    core_axis_name="c", subcore_axis_name="s", num_cores=NC, num_subcores=NS
)


def _sc_agg_body(with_deg, *refs):
    if with_deg:
        (zs, srcr, dstr, zeros_a, zeros_d, ones_h, outp, outd,
         cbuf, acc, ones_v, dbuf, dacc) = refs[:13]
        rest = refs[13:]
    else:
        (zs, srcr, dstr, zeros_a, outp, cbuf, acc) = refs[:7]
        rest = refs[7:]
    sib = rest[:NIB]
    dib = rest[NIB:2 * NIB]
    rows = rest[2 * NIB:2 * NIB + NBUF]
    ssem = rest[2 * NIB + NBUF:3 * NIB + NBUF]
    dsem = rest[3 * NIB + NBUF:4 * NIB + NBUF]
    gsem = rest[4 * NIB + NBUF:]

    c = lax.axis_index("c")
    s = lax.axis_index("s")

    tbl = zs.at[pl.ds(c, 2 * N - 1)]
    srcs = srcr.at[s]
    dsts = dstr.at[s]
    for ib in range(NIB):
        pltpu.make_async_copy(srcs.at[ib], sib[ib], ssem[ib]).start()
        pltpu.make_async_copy(dsts.at[ib], dib[ib], dsem[ib]).start()
    for b in range(NBUF):
        pltpu.make_async_copy(srcs.at[b], sib[b], ssem[b]).wait()
        pltpu.make_async_copy(tbl.at[sib[b]], rows[b], gsem[b]).start()

    pltpu.sync_copy(zeros_a, cbuf)
    for t in range(NCO):
        pltpu.sync_copy(cbuf, acc.at[pl.ds(s * RPT + t * CO, CO)])
    if with_deg:
        pltpu.sync_copy(ones_h, ones_v)
        pltpu.sync_copy(zeros_d, dbuf)
        pltpu.sync_copy(dbuf, dacc.at[pl.ds(s * RPT, RPT)])
    plsc.subcore_barrier()

    def group(g, carry):
        for k in range(NIB):
            j = g * NIB + k
            b = k % NBUF
            pltpu.make_async_copy(
                tbl.at[sib[k]], rows[b], gsem[b]).wait()
            pltpu.make_async_copy(dsts.at[j], dib[k], dsem[k]).wait()
            pltpu.sync_copy(rows[b], acc.at[dib[k]], add=True)
            if with_deg:
                @pl.when(c == (k % 2))
                def _():
                    pltpu.sync_copy(ones_v, dacc.at[dib[k]], add=True)

            @pl.when(g < NGRP - 1)
            def _():
                pltpu.make_async_copy(
                    srcs.at[j + NIB], sib[k], ssem[k]).start()
                pltpu.make_async_copy(
                    dsts.at[j + NIB], dib[k], dsem[k]).start()

            kn = (k + NBUF) % NIB

            @pl.when(j + NBUF < NCH)
            def _():
                pltpu.make_async_copy(
                    srcs.at[j + NBUF], sib[kn], ssem[kn]).wait()
                pltpu.make_async_copy(
                    tbl.at[sib[kn]], rows[b], gsem[b]).start()
        return carry

    lax.fori_loop(0, NGRP, group, 0)
    plsc.subcore_barrier()

    for t in range(NCO):
        row0 = s * RPT + t * CO
        pltpu.sync_copy(acc.at[pl.ds(row0, CO)], cbuf)
        pltpu.sync_copy(cbuf, outp.at[pl.ds(row0, CO), pl.ds(c * H, H)])
    if with_deg:
        pltpu.sync_copy(dacc.at[pl.ds(s * RPT, RPT)], dbuf)
        pltpu.sync_copy(dbuf, outd.at[c, pl.ds(s * RPT, RPT)])


_ring_scratch = (
    [pltpu.VMEM((C,), jnp.int32)] * (2 * NIB)
    + [pltpu.VMEM((C, H), jnp.bfloat16)] * NBUF
    + [pltpu.SemaphoreType.DMA] * (2 * NIB + NBUF)
)

_common_scratch = [
    pltpu.VMEM((CO, H), jnp.bfloat16),
    pltpu.VMEM_SHARED((NPAD, H), jnp.bfloat16),
]


def _agg_deg_body(*refs):
    _sc_agg_body(True, *refs)


def _agg_body(*refs):
    _sc_agg_body(False, *refs)


_sc_params = pltpu.CompilerParams(use_tc_tiling_on_sc=False)


_agg_deg = pl.kernel(
    _agg_deg_body,
    compiler_params=_sc_params,
    out_type=[
        jax.ShapeDtypeStruct((NPAD, D), jnp.bfloat16),
        jax.ShapeDtypeStruct((NC, NPAD, 16), jnp.float32),
    ],
    mesh=_mesh,
    scratch_types=_common_scratch + [
        pltpu.VMEM((C, 16), jnp.float32),
        pltpu.VMEM((RPT, 16), jnp.float32),
        pltpu.VMEM_SHARED((NPAD, 16), jnp.float32),
    ] + _ring_scratch,
)

_agg = pl.kernel(
    _agg_body,
    compiler_params=_sc_params,
    out_type=jax.ShapeDtypeStruct((NPAD, D), jnp.bfloat16),
    mesh=_mesh,
    scratch_types=_common_scratch + _ring_scratch,
)


BR = 2000


def _dense_body(z_ref, p_ref, d0_ref, d1_ref, w_ref, *o_refs):
    deg = jnp.maximum((d0_ref[...] + d1_ref[...])[:, 0:1], 1.0)
    zn = p_ref[...].astype(jnp.float32) / deg
    zl = z_ref[...]
    wl = w_ref[:, :D]
    wr = w_ref[:, D:]
    dn = (((1,), (1,)), ((), ()))
    acc = lax.dot_general(zl, wl, dn, preferred_element_type=jnp.float32)
    acc = acc + lax.dot_general(zn, wr, dn, preferred_element_type=jnp.float32)
    h = jax.nn.sigmoid(acc)
    norm = jnp.sqrt(jnp.sum(h * h, axis=1, keepdims=True)) + 1e-12
    zout = h / norm
    o_refs[0][...] = zout
    if len(o_refs) > 1:
        o_refs[1][...] = zout.astype(jnp.bfloat16)


def _dense(z, p, d, W, bf_out):
    def body(z_ref, p_ref, d0_ref, d1_ref, w_ref, *o_refs):
        _dense_body(z_ref, p_ref, d0_ref.at[0], d1_ref.at[0], w_ref, *o_refs)

    out_specs = [pl.BlockSpec((BR, D), lambda i: (i, 0))]
    out_shape = [jax.ShapeDtypeStruct((N, D), jnp.float32)]
    if bf_out:
        out_specs.append(pl.BlockSpec((BR, D), lambda i: (i, 0)))
        out_shape.append(jax.ShapeDtypeStruct((N, D), jnp.bfloat16))
    return pl.pallas_call(
        body,
        grid=(N // BR,),
        in_specs=[
            pl.BlockSpec((BR, D), lambda i: (i, 0)),
            pl.BlockSpec((BR, D), lambda i: (i, 0)),
            pl.BlockSpec((1, BR, 16), lambda i: (0, i, 0)),
            pl.BlockSpec((1, BR, 16), lambda i: (1, i, 0)),
            pl.BlockSpec((D, 2 * D), lambda i: (0, 0)),
        ],
        out_specs=out_specs,
        out_shape=out_shape,
        compiler_params=pltpu.CompilerParams(
            dimension_semantics=("parallel",)),
    )(z, p, d, d, W)


@jax.jit
def kernel(x, edge_index, W1, W2):
    srcr = (edge_index[0].astype(jnp.int32) * 2).reshape(NS, NCH, C)
    dstr = edge_index[1].astype(jnp.int32).reshape(NS, NCH, C)
    zeros_a = jnp.zeros((CO, H), jnp.bfloat16)
    zeros_d = jnp.zeros((RPT, 16), jnp.float32)
    ones_h = jnp.ones((C, 16), jnp.float32)

    xflat = x.astype(jnp.bfloat16).reshape(2 * N, H)
    p, dp = _agg_deg(xflat, srcr, dstr, zeros_a, zeros_d, ones_h)
    z1, z1b = _dense(x, p, dp, W1, True)
    p2 = _agg(z1b.reshape(2 * N, H), srcr, dstr, zeros_a)
    (z2,) = _dense(z1, p2, dp, W2, False)
    return z2


# speedup vs baseline: 3.6726x; 1.0053x over previous
import jax
import jax.numpy as jnp
from jax import lax
from jax.experimental import pallas as pl
from jax.experimental.pallas import tpu as pltpu, tpu_sc as plsc

N = 10000
D = 128
H = D // 2
E = 320000

NC = 2
NS = 16
NW = NC * NS

C = 125
NCH = E // NS // C
NBUF = 4
NIB = 8
NGRP = NCH // NIB
NPAD = 10240
RPT = NPAD // NS
CO = 128
NCO = RPT // CO

_mesh = plsc.VectorSubcoreMesh(
    core_axis_name="c", subcore_axis_name="s", num_cores=NC, num_subcores=NS
)


def _sc_agg_body(with_deg, *refs):
    if with_deg:
        (zs, srcr, dstr, zeros_a, zeros_d, ones_h, outp, outd,
         cbuf, acc, ones_v, dbuf, dacc) = refs[:13]
        rest = refs[13:]
    else:
        (zs, srcr, dstr, zeros_a, outp, cbuf, acc) = refs[:7]
        rest = refs[7:]
    sib = rest[:NIB]
    dib = rest[NIB:2 * NIB]
    rows = rest[2 * NIB:2 * NIB + NBUF]
    ssem = rest[2 * NIB + NBUF:3 * NIB + NBUF]
    dsem = rest[3 * NIB + NBUF:4 * NIB + NBUF]
    gsem = rest[4 * NIB + NBUF:]

    c = lax.axis_index("c")
    s = lax.axis_index("s")

    tbl = zs.at[pl.ds(c, 2 * N - 1)]
    srcs = srcr.at[s]
    dsts = dstr.at[s]
    for ib in range(NIB):
        pltpu.make_async_copy(srcs.at[ib], sib[ib], ssem[ib]).start()
        pltpu.make_async_copy(dsts.at[ib], dib[ib], dsem[ib]).start()
    for b in range(NBUF):
        pltpu.make_async_copy(srcs.at[b], sib[b], ssem[b]).wait()
        pltpu.make_async_copy(tbl.at[sib[b]], rows[b], gsem[b]).start()

    pltpu.sync_copy(zeros_a, cbuf)
    for t in range(NCO):
        pltpu.sync_copy(cbuf, acc.at[pl.ds(s * RPT + t * CO, CO)])
    if with_deg:
        pltpu.sync_copy(ones_h, ones_v)
        pltpu.sync_copy(zeros_d, dbuf)
        pltpu.sync_copy(dbuf, dacc.at[pl.ds(s * RPT, RPT)])
    plsc.subcore_barrier()

    def group(g, carry):
        for k in range(NIB):
            j = g * NIB + k
            b = k % NBUF
            pltpu.make_async_copy(
                tbl.at[sib[k]], rows[b], gsem[b]).wait()
            pltpu.make_async_copy(dsts.at[j], dib[k], dsem[k]).wait()
            pltpu.sync_copy(rows[b], acc.at[dib[k]], add=True)
            if with_deg:
                @pl.when(c == (k % 2))
                def _():
                    pltpu.sync_copy(ones_v, dacc.at[dib[k]], add=True)

            @pl.when(g < NGRP - 1)
            def _():
                pltpu.make_async_copy(
                    srcs.at[j + NIB], sib[k], ssem[k]).start()
                pltpu.make_async_copy(
                    dsts.at[j + NIB], dib[k], dsem[k]).start()

            kn = (k + NBUF) % NIB

            @pl.when(j + NBUF < NCH)
            def _():
                pltpu.make_async_copy(
                    srcs.at[j + NBUF], sib[kn], ssem[kn]).wait()
                pltpu.make_async_copy(
                    tbl.at[sib[kn]], rows[b], gsem[b]).start()
        return carry

    lax.fori_loop(0, NGRP, group, 0)
    plsc.subcore_barrier()

    for t in range(NCO):
        row0 = s * RPT + t * CO
        pltpu.sync_copy(acc.at[pl.ds(row0, CO)], cbuf)
        pltpu.sync_copy(cbuf, outp.at[pl.ds(row0, CO), pl.ds(c * H, H)])
    if with_deg:
        pltpu.sync_copy(dacc.at[pl.ds(s * RPT, RPT)], dbuf)
        pltpu.sync_copy(dbuf, outd.at[c, pl.ds(s * RPT, RPT)])


_ring_scratch = (
    [pltpu.VMEM((C,), jnp.int32)] * (2 * NIB)
    + [pltpu.VMEM((C, H), jnp.bfloat16)] * NBUF
    + [pltpu.SemaphoreType.DMA] * (2 * NIB + NBUF)
)

_common_scratch = [
    pltpu.VMEM((CO, H), jnp.bfloat16),
    pltpu.VMEM_SHARED((NPAD, H), jnp.bfloat16),
]


def _agg_deg_body(*refs):
    _sc_agg_body(True, *refs)


def _agg_body(*refs):
    _sc_agg_body(False, *refs)


_sc_params = pltpu.CompilerParams(use_tc_tiling_on_sc=False)


_agg_deg = pl.kernel(
    _agg_deg_body,
    compiler_params=_sc_params,
    out_type=[
        jax.ShapeDtypeStruct((NPAD, D), jnp.bfloat16),
        jax.ShapeDtypeStruct((NC, NPAD, 8), jnp.float32),
    ],
    mesh=_mesh,
    scratch_types=_common_scratch + [
        pltpu.VMEM((C, 8), jnp.float32),
        pltpu.VMEM((RPT, 8), jnp.float32),
        pltpu.VMEM_SHARED((NPAD, 8), jnp.float32),
    ] + _ring_scratch,
)

_agg = pl.kernel(
    _agg_body,
    compiler_params=_sc_params,
    out_type=jax.ShapeDtypeStruct((NPAD, D), jnp.bfloat16),
    mesh=_mesh,
    scratch_types=_common_scratch + _ring_scratch,
)


BR = 2000


def _dense_body(z_ref, p_ref, d0_ref, d1_ref, w_ref, *o_refs):
    deg = jnp.maximum((d0_ref[...] + d1_ref[...])[:, 0:1], 1.0)
    zn = p_ref[...].astype(jnp.float32) / deg
    zl = z_ref[...]
    wl = w_ref[:, :D]
    wr = w_ref[:, D:]
    dn = (((1,), (1,)), ((), ()))
    acc = lax.dot_general(zl, wl, dn, preferred_element_type=jnp.float32)
    acc = acc + lax.dot_general(zn, wr, dn, preferred_element_type=jnp.float32)
    h = jax.nn.sigmoid(acc)
    norm = jnp.sqrt(jnp.sum(h * h, axis=1, keepdims=True)) + 1e-12
    zout = h / norm
    o_refs[0][...] = zout
    if len(o_refs) > 1:
        o_refs[1][...] = zout.astype(jnp.bfloat16)


def _dense(z, p, d, W, bf_out):
    def body(z_ref, p_ref, d0_ref, d1_ref, w_ref, *o_refs):
        _dense_body(z_ref, p_ref, d0_ref.at[0], d1_ref.at[0], w_ref, *o_refs)

    out_specs = [pl.BlockSpec((BR, D), lambda i: (i, 0))]
    out_shape = [jax.ShapeDtypeStruct((N, D), jnp.float32)]
    if bf_out:
        out_specs.append(pl.BlockSpec((BR, D), lambda i: (i, 0)))
        out_shape.append(jax.ShapeDtypeStruct((N, D), jnp.bfloat16))
    return pl.pallas_call(
        body,
        grid=(N // BR,),
        in_specs=[
            pl.BlockSpec((BR, D), lambda i: (i, 0)),
            pl.BlockSpec((BR, D), lambda i: (i, 0)),
            pl.BlockSpec((1, BR, 8), lambda i: (0, i, 0)),
            pl.BlockSpec((1, BR, 8), lambda i: (1, i, 0)),
            pl.BlockSpec((D, 2 * D), lambda i: (0, 0)),
        ],
        out_specs=out_specs,
        out_shape=out_shape,
        compiler_params=pltpu.CompilerParams(
            dimension_semantics=("parallel",)),
    )(z, p, d, d, W)


@jax.jit
def kernel(x, edge_index, W1, W2):
    srcr = (edge_index[0].astype(jnp.int32) * 2).reshape(NS, NCH, C)
    dstr = edge_index[1].astype(jnp.int32).reshape(NS, NCH, C)
    zeros_a = jnp.zeros((CO, H), jnp.bfloat16)
    zeros_d = jnp.zeros((RPT, 8), jnp.float32)
    ones_h = jnp.ones((C, 8), jnp.float32)

    xflat = x.astype(jnp.bfloat16).reshape(2 * N, H)
    p, dp = _agg_deg(xflat, srcr, dstr, zeros_a, zeros_d, ones_h)
    z1, z1b = _dense(x, p, dp, W1, True)
    p2 = _agg(z1b.reshape(2 * N, H), srcr, dstr, zeros_a)
    (z2,) = _dense(z1, p2, dp, W2, False)
    return z2
